# baseline (speedup 1.0000x reference)
import functools

import jax
import jax.numpy as jnp
from jax import lax
from jax.experimental import pallas as pl
from jax.experimental.pallas import tpu as pltpu
from jax.experimental.pallas import tpu_sc as plsc

_NUM_CORES = 2
_NUM_SUBCORES = 16
_NW = _NUM_CORES * _NUM_SUBCORES

_V = 1000000
_D = 32
_K = 1024
_NCHUNK = _V // _K
_TAIL = _V - _NCHUNK * _K
_B = 16384
_C = 26
_BW = _B // _NW


def _mesh():
    return plsc.VectorSubcoreMesh(
        core_axis_name="c", subcore_axis_name="s", num_cores=_NUM_CORES
    )


def _wid():
    return lax.axis_index("s") * _NUM_CORES + lax.axis_index("c")


@functools.lru_cache(maxsize=None)
def _build_relayout():
    @functools.partial(
        pl.kernel,
        mesh=_mesh(),
        out_type=jax.ShapeDtypeStruct((_V * _D,), jnp.float32),
        scratch_types=[
            pltpu.VMEM((_D, _K), jnp.float32),
            pltpu.VMEM((_D * _K,), jnp.float32),
            pltpu.VMEM((_D, _TAIL), jnp.float32),
        ],
        compiler_params=pltpu.CompilerParams(
            use_tc_tiling_on_sc=True, needs_layout_passes=False
        ),
    )
    def relayout(wt_hbm, wlin_hbm, src_v, dst_v, tail_v):
        wid = _wid()
        iota = lax.iota(jnp.int32, 16)
        n_iter = (_NCHUNK + _NW - 1) // _NW

        def chunk_body(n, carry):
            chunk = jnp.minimum(n * _NW + wid, _NCHUNK - 1)
            pltpu.sync_copy(wt_hbm.at[:, pl.ds(chunk * _K, _K)], src_v)

            def tr_body(j, c2):
                jv = j * 16 + iota
                f = lax.rem(jv, _D)
                l = lax.div(jv, _D)
                v = plsc.load_gather(src_v, [f, l])
                dst_v[pl.ds(j * 16, 16)] = v
                return c2

            lax.fori_loop(0, (_D * _K) // 16, tr_body, 0, unroll=4)
            pltpu.sync_copy(dst_v, wlin_hbm.at[pl.ds(chunk * _D * _K, _D * _K)])
            return carry

        lax.fori_loop(0, n_iter, chunk_body, 0)

        @pl.when(wid == 0)
        def _tail():
            base = _NCHUNK * _K
            pltpu.sync_copy(wt_hbm.at[:, pl.ds(base, _TAIL)], tail_v)

            def tr_body(j, c2):
                jv = j * 16 + iota
                f = lax.rem(jv, _D)
                l = lax.div(jv, _D)
                v = plsc.load_gather(tail_v, [f, l])
                dst_v[pl.ds(j * 16, 16)] = v
                return c2

            lax.fori_loop(0, (_D * _TAIL) // 16, tr_body, 0, unroll=4)
            pltpu.sync_copy(
                dst_v.at[pl.ds(0, _D * _TAIL)],
                wlin_hbm.at[pl.ds(base * _D, _D * _TAIL)],
            )

    return relayout


@functools.lru_cache(maxsize=None)
def _build_embed():
    @functools.partial(
        pl.kernel,
        mesh=_mesh(),
        out_type=jax.ShapeDtypeStruct((_C * _D * _B,), jnp.float32),
        scratch_types=[
            pltpu.VMEM((_BW,), jnp.int32),
            pltpu.VMEM((_BW, _D), jnp.float32),
            pltpu.VMEM((_BW * _D,), jnp.float32),
            pltpu.SemaphoreType.DMA,
        ],
        compiler_params=pltpu.CompilerParams(
            use_tc_tiling_on_sc=False, needs_layout_passes=False
        ),
    )
    def embed(wlin_hbm, idx_hbm, out_hbm, idx_v, rows_v, obuf_v, gsem):
        wid = _wid()
        iota = lax.iota(jnp.int32, 16)

        def col_body(c, carry):
            pltpu.sync_copy(idx_hbm.at[pl.ds(c * _B + wid * _BW, _BW)], idx_v)
            pltpu.async_copy(wlin_hbm.at[idx_v], rows_v, gsem).wait()

            def tr_body(j, c2):
                jv = j * 16 + iota
                bl = jnp.bitwise_and(jv, 127)
                fs = jnp.bitwise_and(lax.shift_right_logical(jv, 7), 7)
                bkr = jnp.bitwise_and(lax.shift_right_logical(jv, 10), 3)
                ft = lax.shift_right_logical(jv, 12)
                r = bkr * 128 + bl
                f = ft * 8 + fs
                v = plsc.load_gather(rows_v, [r, f])
                obuf_v[pl.ds(j * 16, 16)] = v
                return c2

            lax.fori_loop(0, (_BW * _D) // 16, tr_body, 0, unroll=4)
            for ft in range(4):
                pltpu.sync_copy(
                    obuf_v.at[pl.ds(ft * 4096, 4096)],
                    out_hbm.at[pl.ds(c * (_D * _B) + ft * (8 * _B) + wid * 4096, 4096)],
                )
            return carry

        lax.fori_loop(0, _C, col_body, 0)

    return embed


def kernel(x, W):
    w_t = W.T
    w_lin = _build_relayout()(w_t)
    idx = x.T.reshape(-1).astype(jnp.int32)
    out1d = _build_embed()(w_lin.reshape(_V, _D), idx)
    out5 = out1d.reshape(_C, 4, 128, 8, 128)
    return out5.transpose(2, 4, 0, 1, 3).reshape(_B, _C, _D)


# speedup vs baseline: 1.2001x; 1.2001x over previous
import functools

import jax
import jax.numpy as jnp
from jax import lax
from jax.experimental import pallas as pl
from jax.experimental.pallas import tpu as pltpu
from jax.experimental.pallas import tpu_sc as plsc

_NUM_CORES = 2
_NUM_SUBCORES = 16
_NW = _NUM_CORES * _NUM_SUBCORES

_V = 1000000
_D = 32
_K = 1024
_NCHUNK = _V // _K
_TAIL = _V - _NCHUNK * _K
_B = 16384
_C = 26
_BW = _B // _NW


def _mesh():
    return plsc.VectorSubcoreMesh(
        core_axis_name="c", subcore_axis_name="s", num_cores=_NUM_CORES
    )


def _wid():
    return lax.axis_index("s") * _NUM_CORES + lax.axis_index("c")


@functools.lru_cache(maxsize=None)
def _build_relayout():
    @functools.partial(
        pl.kernel,
        mesh=_mesh(),
        out_type=jax.ShapeDtypeStruct((_V * _D,), jnp.float32),
        scratch_types=[
            pltpu.VMEM((_D, _K), jnp.float32),
            pltpu.VMEM((_D * _K,), jnp.float32),
            pltpu.VMEM((_D, _TAIL), jnp.float32),
        ],
        compiler_params=pltpu.CompilerParams(
            use_tc_tiling_on_sc=True, needs_layout_passes=False
        ),
    )
    def relayout(wt_hbm, wlin_hbm, src_v, dst_v, tail_v):
        wid = _wid()
        iota = lax.iota(jnp.int32, 16)
        stride32 = iota * _D
        n_iter = (_NCHUNK + _NW - 1) // _NW

        def transpose_chunk(src, n_lanes):
            nblk = n_lanes // 16

            def body(j, c2):
                f = lax.div(j, nblk)
                l0 = lax.rem(j, nblk) * 16
                v = src[f, pl.ds(l0, 16)]
                plsc.store_scatter(dst_v, [l0 * _D + f + stride32], v)
                return c2

            lax.fori_loop(0, _D * nblk, body, 0, unroll=8)

        def chunk_body(n, carry):
            chunk = jnp.minimum(n * _NW + wid, _NCHUNK - 1)
            pltpu.sync_copy(wt_hbm.at[:, pl.ds(chunk * _K, _K)], src_v)
            transpose_chunk(src_v, _K)
            pltpu.sync_copy(dst_v, wlin_hbm.at[pl.ds(chunk * _D * _K, _D * _K)])
            return carry

        lax.fori_loop(0, n_iter, chunk_body, 0)

        @pl.when(wid == 0)
        def _tail():
            base = _NCHUNK * _K
            pltpu.sync_copy(wt_hbm.at[:, pl.ds(base, _TAIL)], tail_v)
            transpose_chunk(tail_v, _TAIL)
            pltpu.sync_copy(
                dst_v.at[pl.ds(0, _D * _TAIL)],
                wlin_hbm.at[pl.ds(base * _D, _D * _TAIL)],
            )

    return relayout


@functools.lru_cache(maxsize=None)
def _build_embed():
    @functools.partial(
        pl.kernel,
        mesh=_mesh(),
        out_type=jax.ShapeDtypeStruct((_C * _D * _B,), jnp.float32),
        scratch_types=[
            pltpu.VMEM((_BW,), jnp.int32),
            pltpu.VMEM((_BW, _D), jnp.float32),
            pltpu.VMEM((_BW * _D,), jnp.float32),
            pltpu.SemaphoreType.DMA,
        ],
        compiler_params=pltpu.CompilerParams(
            use_tc_tiling_on_sc=False, needs_layout_passes=False
        ),
    )
    def embed(wlin_hbm, idx_hbm, out_hbm, idx_v, rows_v, obuf_v, gsem):
        wid = _wid()
        iota = lax.iota(jnp.int32, 16)

        pat0 = lax.shift_right_logical(iota, 3) * 4096 + jnp.bitwise_and(iota, 7) * 128
        pat1 = pat0 + 8192

        def col_body(c, carry):
            pltpu.sync_copy(idx_hbm.at[pl.ds(c * _B + wid * _BW, _BW)], idx_v)
            pltpu.async_copy(wlin_hbm.at[idx_v], rows_v, gsem).wait()

            def tr_body(r, c2):
                base = (
                    lax.shift_right_logical(r, 7) * 1024
                    + jnp.bitwise_and(r, 127)
                )
                v0 = rows_v[r, pl.ds(0, 16)]
                v1 = rows_v[r, pl.ds(16, 16)]
                plsc.store_scatter(obuf_v, [pat0 + base], v0)
                plsc.store_scatter(obuf_v, [pat1 + base], v1)
                return c2

            lax.fori_loop(0, _BW, tr_body, 0, unroll=8)
            for ft in range(4):
                pltpu.sync_copy(
                    obuf_v.at[pl.ds(ft * 4096, 4096)],
                    out_hbm.at[pl.ds(c * (_D * _B) + ft * (8 * _B) + wid * 4096, 4096)],
                )
            return carry

        lax.fori_loop(0, _C, col_body, 0)

    return embed


def kernel(x, W):
    w_t = W.T
    w_lin = _build_relayout()(w_t)
    idx = x.T.reshape(-1).astype(jnp.int32)
    out1d = _build_embed()(w_lin.reshape(_V, _D), idx)
    out5 = out1d.reshape(_C, 4, 128, 8, 128)
    return out5.transpose(2, 4, 0, 1, 3).reshape(_B, _C, _D)


# speedup vs baseline: 2.2786x; 1.8987x over previous
import functools

import jax
import jax.numpy as jnp
from jax import lax
from jax.experimental import pallas as pl
from jax.experimental.pallas import tpu as pltpu
from jax.experimental.pallas import tpu_sc as plsc

_NUM_CORES = 2
_NUM_SUBCORES = 16
_NW = _NUM_CORES * _NUM_SUBCORES

_V = 1000000
_D = 32
_K = 1024
_NCHUNK = _V // _K
_TAIL = _V - _NCHUNK * _K
_B = 16384
_C = 26
_BW = _B // _NW


def _mesh():
    return plsc.VectorSubcoreMesh(
        core_axis_name="c", subcore_axis_name="s", num_cores=_NUM_CORES
    )


def _wid():
    return lax.axis_index("s") * _NUM_CORES + lax.axis_index("c")


@functools.lru_cache(maxsize=None)
def _build_relayout():
    @functools.partial(
        pl.kernel,
        mesh=_mesh(),
        out_type=jax.ShapeDtypeStruct((_V * _D,), jnp.float32),
        scratch_types=[
            pltpu.VMEM((_D, _K), jnp.float32),
            pltpu.VMEM((_D * _K,), jnp.float32),
            pltpu.VMEM((_D, _TAIL), jnp.float32),
        ],
        compiler_params=pltpu.CompilerParams(
            use_tc_tiling_on_sc=True, needs_layout_passes=False
        ),
    )
    def relayout(wt_hbm, wlin_hbm, src_v, dst_v, tail_v):
        wid = _wid()
        iota = lax.iota(jnp.int32, 16)
        stride32 = iota * _D
        n_iter = (_NCHUNK + _NW - 1) // _NW

        def transpose_chunk(src, n_lanes):
            nblk = n_lanes // 16

            def body(j, c2):
                f0 = jnp.bitwise_and(j, _D - 1)
                l0 = lax.shift_right_logical(j, 5) * 16
                f = jnp.bitwise_and(f0 + iota, _D - 1)
                l = l0 + iota
                v = plsc.load_gather(src, [f, l])
                plsc.store_scatter(dst_v, [l0 * _D + stride32 + f], v)
                return c2

            lax.fori_loop(0, _D * nblk, body, 0, unroll=8)

        def chunk_body(n, carry):
            chunk = jnp.minimum(n * _NW + wid, _NCHUNK - 1)
            pltpu.sync_copy(wt_hbm.at[:, pl.ds(chunk * _K, _K)], src_v)
            transpose_chunk(src_v, _K)
            pltpu.sync_copy(dst_v, wlin_hbm.at[pl.ds(chunk * _D * _K, _D * _K)])
            return carry

        lax.fori_loop(0, n_iter, chunk_body, 0)

        @pl.when(wid == 0)
        def _tail():
            base = _NCHUNK * _K
            pltpu.sync_copy(wt_hbm.at[:, pl.ds(base, _TAIL)], tail_v)
            transpose_chunk(tail_v, _TAIL)
            pltpu.sync_copy(
                dst_v.at[pl.ds(0, _D * _TAIL)],
                wlin_hbm.at[pl.ds(base * _D, _D * _TAIL)],
            )

    return relayout


@functools.lru_cache(maxsize=None)
def _build_embed():
    @functools.partial(
        pl.kernel,
        mesh=_mesh(),
        out_type=jax.ShapeDtypeStruct((_C * _D * _B,), jnp.float32),
        scratch_types=[
            pltpu.VMEM((_BW,), jnp.int32),
            pltpu.VMEM((_BW, _D), jnp.float32),
            pltpu.VMEM((_BW * _D,), jnp.float32),
            pltpu.SemaphoreType.DMA,
        ],
        compiler_params=pltpu.CompilerParams(
            use_tc_tiling_on_sc=False, needs_layout_passes=False
        ),
    )
    def embed(wlin_hbm, idx_hbm, out_hbm, idx_v, rows_v, obuf_v, gsem):
        wid = _wid()
        iota = lax.iota(jnp.int32, 16)

        def col_body(c, carry):
            pltpu.sync_copy(idx_hbm.at[pl.ds(c * _B + wid * _BW, _BW)], idx_v)
            pltpu.async_copy(wlin_hbm.at[idx_v], rows_v, gsem).wait()

            def tr_body(j, c2):
                f0 = jnp.bitwise_and(j, 31)
                r0 = lax.shift_right_logical(j, 5) * 16
                f = jnp.bitwise_and(f0 + iota, 31)
                r = r0 + iota
                v = plsc.load_gather(rows_v, [r, f])
                rb = (
                    lax.shift_right_logical(r0, 7) * 1024
                    + jnp.bitwise_and(r0, 127)
                )
                dst = (
                    lax.shift_right_logical(f, 3) * 4096
                    + jnp.bitwise_and(f, 7) * 128
                    + rb
                    + iota
                )
                plsc.store_scatter(obuf_v, [dst], v)
                return c2

            lax.fori_loop(0, (_BW * _D) // 16, tr_body, 0, unroll=8)
            for ft in range(4):
                pltpu.sync_copy(
                    obuf_v.at[pl.ds(ft * 4096, 4096)],
                    out_hbm.at[pl.ds(c * (_D * _B) + ft * (8 * _B) + wid * 4096, 4096)],
                )
            return carry

        lax.fori_loop(0, _C, col_body, 0)

    return embed


def kernel(x, W):
    w_t = W.T
    w_lin = _build_relayout()(w_t)
    idx = x.T.reshape(-1).astype(jnp.int32)
    out1d = _build_embed()(w_lin.reshape(_V, _D), idx)
    out5 = out1d.reshape(_C, 4, 128, 8, 128)
    return out5.transpose(2, 4, 0, 1, 3).reshape(_B, _C, _D)


# speedup vs baseline: 3.3028x; 1.4495x over previous
import functools

import jax
import jax.numpy as jnp
from jax import lax
from jax.experimental import pallas as pl
from jax.experimental.pallas import tpu as pltpu
from jax.experimental.pallas import tpu_sc as plsc

_NUM_CORES = 2
_NUM_SUBCORES = 16
_NW = _NUM_CORES * _NUM_SUBCORES

_V = 1000000
_D = 32
_K = 512
_NCHUNK = _V // _K
_TAIL = _V - _NCHUNK * _K
_B = 16384
_C = 26
_BW = _B // _NW


def _mesh():
    return plsc.VectorSubcoreMesh(
        core_axis_name="c", subcore_axis_name="s", num_cores=_NUM_CORES
    )


def _wid():
    return lax.axis_index("s") * _NUM_CORES + lax.axis_index("c")


@functools.lru_cache(maxsize=None)
def _build_relayout():
    n_iter = (_NCHUNK + _NW - 1) // _NW
    assert n_iter % 2 == 0

    @functools.partial(
        pl.kernel,
        mesh=_mesh(),
        out_type=jax.ShapeDtypeStruct((_V * _D,), jnp.float32),
        scratch_types=[
            pltpu.VMEM((_D, _K), jnp.float32),
            pltpu.VMEM((_D, _K), jnp.float32),
            pltpu.VMEM((_D * _K,), jnp.float32),
            pltpu.VMEM((_D * _K,), jnp.float32),
            pltpu.VMEM((_D, _TAIL), jnp.float32),
            pltpu.SemaphoreType.DMA,
            pltpu.SemaphoreType.DMA,
            pltpu.SemaphoreType.DMA,
            pltpu.SemaphoreType.DMA,
        ],
        compiler_params=pltpu.CompilerParams(
            use_tc_tiling_on_sc=True, needs_layout_passes=False
        ),
    )
    def relayout(wt_hbm, wlin_hbm, src0_v, src1_v, dst0_v, dst1_v, tail_v, l0s, l1s, w0s, w1s):
        wid = _wid()
        iota = lax.iota(jnp.int32, 16)
        stride32 = iota * _D
        lsem = (l0s, l1s)
        wsem = (w0s, w1s)
        srcs = (src0_v, src1_v)
        dsts = (dst0_v, dst1_v)

        def chunk_of(k):
            return jnp.minimum(k * _NW + wid, _NCHUNK - 1)

        def start_load(k, b):
            pltpu.async_copy(
                wt_hbm.at[:, pl.ds(chunk_of(k) * _K, _K)], srcs[b], lsem[b]
            )

        def transpose(src, dst, n_lanes):
            def body(j, c2):
                f0 = jnp.bitwise_and(j, _D - 1)
                l0 = lax.shift_right_logical(j, 5) * 16
                f = jnp.bitwise_and(f0 + iota, _D - 1)
                l = l0 + iota
                v = plsc.load_gather(src, [f, l])
                plsc.store_scatter(dst, [l0 * _D + stride32 + f], v)
                return c2

            lax.fori_loop(0, _D * (n_lanes // 16), body, 0, unroll=8)

        start_load(0, 0)
        start_load(1, 1)

        def outer(t, carry):
            for b in range(2):
                k = 2 * t + b
                pltpu.make_async_copy(
                    wt_hbm.at[:, pl.ds(chunk_of(k) * _K, _K)],
                    srcs[b],
                    lsem[b],
                ).wait()

                @pl.when(k >= 2)
                def _drain():
                    pltpu.make_async_copy(
                        dsts[b],
                        wlin_hbm.at[pl.ds(chunk_of(k - 2) * _D * _K, _D * _K)],
                        wsem[b],
                    ).wait()

                transpose(srcs[b], dsts[b], _K)
                pltpu.async_copy(
                    dsts[b],
                    wlin_hbm.at[pl.ds(chunk_of(k) * _D * _K, _D * _K)],
                    wsem[b],
                )

                @pl.when(k + 2 < n_iter)
                def _next():
                    start_load(k + 2, b)

            return carry

        lax.fori_loop(0, n_iter // 2, outer, 0)
        for b in range(2):
            pltpu.make_async_copy(
                dsts[b],
                wlin_hbm.at[pl.ds(chunk_of(n_iter - 2 + b) * _D * _K, _D * _K)],
                wsem[b],
            ).wait()

        @pl.when(wid == 0)
        def _tail():
            base = _NCHUNK * _K
            pltpu.sync_copy(wt_hbm.at[:, pl.ds(base, _TAIL)], tail_v)
            transpose(tail_v, dst0_v, _TAIL)
            pltpu.sync_copy(
                dst0_v.at[pl.ds(0, _D * _TAIL)],
                wlin_hbm.at[pl.ds(base * _D, _D * _TAIL)],
            )

    return relayout


@functools.lru_cache(maxsize=None)
def _build_embed():
    assert _C % 2 == 0

    @functools.partial(
        pl.kernel,
        mesh=_mesh(),
        out_type=jax.ShapeDtypeStruct((_C * _D * _B,), jnp.float32),
        scratch_types=[
            pltpu.VMEM((_C, _BW), jnp.int32),
            pltpu.VMEM((_BW, _D), jnp.float32),
            pltpu.VMEM((_BW, _D), jnp.float32),
            pltpu.VMEM((_BW * _D,), jnp.float32),
            pltpu.VMEM((_BW * _D,), jnp.float32),
            pltpu.SemaphoreType.DMA,
            pltpu.SemaphoreType.DMA,
            pltpu.SemaphoreType.DMA,
            pltpu.SemaphoreType.DMA,
        ],
        compiler_params=pltpu.CompilerParams(
            use_tc_tiling_on_sc=False, needs_layout_passes=False
        ),
    )
    def embed(wlin_hbm, idx_hbm, out_hbm, idx_v, rows0_v, rows1_v, obuf0_v, obuf1_v, g0, g1, o0, o1):
        wid = _wid()
        iota = lax.iota(jnp.int32, 16)
        gsem = (g0, g1)
        osem = (o0, o1)
        rows = (rows0_v, rows1_v)
        obufs = (obuf0_v, obuf1_v)

        pltpu.sync_copy(idx_hbm.at[:, pl.ds(wid * _BW, _BW)], idx_v)

        def start_gather(c, b):
            pltpu.async_copy(wlin_hbm.at[idx_v.at[c]], rows[b], gsem[b])

        def store_out(c, b):
            for ft in range(4):
                pltpu.async_copy(
                    obufs[b].at[pl.ds(ft * 4096, 4096)],
                    out_hbm.at[
                        pl.ds(c * (_D * _B) + ft * (8 * _B) + wid * 4096, 4096)
                    ],
                    osem[b],
                )

        def drain_out(c, b):
            for ft in range(4):
                pltpu.make_async_copy(
                    obufs[b].at[pl.ds(ft * 4096, 4096)],
                    out_hbm.at[
                        pl.ds(c * (_D * _B) + ft * (8 * _B) + wid * 4096, 4096)
                    ],
                    osem[b],
                ).wait()

        start_gather(0, 0)
        start_gather(1, 1)

        def outer(t, carry):
            for b in range(2):
                c = 2 * t + b
                pltpu.make_async_copy(
                    wlin_hbm.at[idx_v.at[c]], rows[b], gsem[b]
                ).wait()

                @pl.when(c >= 2)
                def _drain():
                    drain_out(c - 2, b)

                def tr_body(j, c2):
                    f0 = jnp.bitwise_and(j, 31)
                    r0 = lax.shift_right_logical(j, 5) * 16
                    f = jnp.bitwise_and(f0 + iota, 31)
                    r = r0 + iota
                    v = plsc.load_gather(rows[b], [r, f])
                    rb = (
                        lax.shift_right_logical(r0, 7) * 1024
                        + jnp.bitwise_and(r0, 127)
                    )
                    dst = (
                        lax.shift_right_logical(f, 3) * 4096
                        + jnp.bitwise_and(f, 7) * 128
                        + rb
                        + iota
                    )
                    plsc.store_scatter(obufs[b], [dst], v)
                    return c2

                lax.fori_loop(0, (_BW * _D) // 16, tr_body, 0, unroll=8)
                store_out(c, b)

                @pl.when(c + 2 < _C)
                def _next():
                    start_gather(c + 2, b)

            return carry

        lax.fori_loop(0, _C // 2, outer, 0)
        for b in range(2):
            drain_out(_C - 2 + b, b)

    return embed


def kernel(x, W):
    w_t = W.T
    w_lin = _build_relayout()(w_t)
    idx2 = x.T.astype(jnp.int32)
    out1d = _build_embed()(w_lin.reshape(_V, _D), idx2)
    out5 = out1d.reshape(_C, 4, 128, 8, 128)
    return out5.transpose(2, 4, 0, 1, 3).reshape(_B, _C, _D)
